# Initial kernel scaffold; baseline (speedup 1.0000x reference)
import jax
import jax.numpy as jnp
from jax.experimental import pallas as pl


def kernel(x, Wr, br, W1, b1, W2, b2, W3, b3):
    raise NotImplementedError("write your pallas kernel here")


# speedup vs baseline: 2.6656x; 2.6656x over previous
import functools

import jax
import jax.numpy as jnp
from jax import lax
from jax.experimental import pallas as pl
from jax.experimental.pallas import tpu as pltpu

S, H, F, E, K = 2048, 768, 3072, 8, 2
F2 = F // 2
EP = 128
TS = 256
NT_DENSE = S // TS
NEG = -1e30


def _gelu_exact(h):
    return 0.5 * h * (1.0 + lax.erf(h * 0.7071067811865476))


def _routing_body(x_ref, wr_ref, br_ref, w2_ref, sel2_ref, div_ref, wsel_ref, pos_ref):
    xb = x_ref[...]
    logits = lax.dot_general(
        xb, wr_ref[...], (((1,), (0,)), ((), ())),
        preferred_element_type=jnp.float32)
    logits = logits + br_ref[0:1, :]

    lane = lax.broadcasted_iota(jnp.int32, (S, EP), 1)
    v0 = jnp.max(logits, axis=1, keepdims=True)
    e0 = jnp.min(jnp.where(logits == v0, lane, EP), axis=1, keepdims=True)
    l1 = jnp.where(lane == e0, NEG, logits)
    v1 = jnp.max(l1, axis=1, keepdims=True)
    e1 = jnp.min(jnp.where(l1 == v1, lane, EP), axis=1, keepdims=True)

    w0 = 1.0 / (1.0 + jnp.exp(v1 - v0))
    w1 = 1.0 - w0

    sel2_ref[...] = jnp.concatenate([e0, e1], axis=1)
    w2_ref[...] = jnp.concatenate([w0, w1], axis=1)

    p = jnp.exp(logits - v0)
    p = p / jnp.sum(p, axis=1, keepdims=True)
    avg = jnp.mean(p, axis=0, keepdims=True)
    ent = -jnp.sum(avg * jnp.log(avg + 1e-8))
    max_ent = jnp.log(float(E))
    div_ref[0, 0] = (max_ent - ent) / max_ent

    oh0 = (lane == e0)
    oh1 = (lane == e1)
    wsel_ref[...] = jnp.where(oh0, w0, 0.0) + jnp.where(oh1, w1, 0.0)

    oh = jnp.concatenate([oh0.astype(jnp.bfloat16), oh1.astype(jnp.bfloat16)], axis=0)
    r_i = lax.broadcasted_iota(jnp.int32, (TS, TS), 0)
    c_i = lax.broadcasted_iota(jnp.int32, (TS, TS), 1)
    tri = (c_i <= r_i).astype(jnp.bfloat16)
    nch = (2 * S) // TS
    cums = []
    tots = []
    for c in range(nch):
        ohc = oh[c * TS:(c + 1) * TS, :]
        cw = lax.dot_general(tri, ohc, (((1,), (0,)), ((), ())),
                             preferred_element_type=jnp.float32)
        cums.append(cw)
        tots.append(cw[TS - 1:TS, :])
    tot = jnp.concatenate(tots, axis=0)
    r16 = lax.broadcasted_iota(jnp.int32, (nch, nch), 0)
    c16 = lax.broadcasted_iota(jnp.int32, (nch, nch), 1)
    tri16 = (c16 < r16).astype(jnp.bfloat16)
    choff = lax.dot_general(tri16, tot.astype(jnp.bfloat16), (((1,), (0,)), ((), ())),
                            preferred_element_type=jnp.float32)
    counts = jnp.sum(tot, axis=0, keepdims=True)
    padded = jnp.floor((counts + (TS - 1.0)) / TS) * TS
    rU = lax.broadcasted_iota(jnp.int32, (EP, EP), 0)
    cU = lax.broadcasted_iota(jnp.int32, (EP, EP), 1)
    U = (rU < cU).astype(jnp.bfloat16)
    offs = lax.dot_general(padded.astype(jnp.bfloat16), U, (((1,), (0,)), ((), ())),
                           preferred_element_type=jnp.float32)

    csum = jnp.concatenate(
        [cums[c] + choff[c:c + 1, :] for c in range(nch)], axis=0)
    ohf = oh.astype(jnp.float32)
    rank = jnp.sum(ohf * csum, axis=1, keepdims=True) - 1.0
    ofa = jnp.sum(ohf * offs, axis=1, keepdims=True)
    pos_ref[...] = (ofa + rank).astype(jnp.int32)


def _routing(x2d, wrp, brp):
    return pl.pallas_call(
        _routing_body,
        out_shape=(
            jax.ShapeDtypeStruct((S, K), jnp.float32),
            jax.ShapeDtypeStruct((S, K), jnp.int32),
            jax.ShapeDtypeStruct((1, 1), jnp.float32),
            jax.ShapeDtypeStruct((S, EP), jnp.float32),
            jax.ShapeDtypeStruct((2 * S, 1), jnp.int32),
        ),
        out_specs=(
            pl.BlockSpec(memory_space=pltpu.VMEM),
            pl.BlockSpec(memory_space=pltpu.VMEM),
            pl.BlockSpec(memory_space=pltpu.SMEM),
            pl.BlockSpec(memory_space=pltpu.VMEM),
            pl.BlockSpec(memory_space=pltpu.VMEM),
        ),
    )(x2d, wrp, brp)


def _dense_body(x_ref, w1_ref, b1_ref, w2_ref, b2_ref, w3_ref, b3_ref, wsel_ref,
                out_ref):
    e = pl.program_id(0)
    t = pl.program_id(1)
    xb = x_ref[...].astype(jnp.bfloat16)
    h = lax.dot_general(xb, w1_ref[0], (((1,), (0,)), ((), ())),
                        preferred_element_type=jnp.float32) + b1_ref[0]
    h = _gelu_exact(h).astype(jnp.bfloat16)
    h2 = lax.dot_general(h, w2_ref[0], (((1,), (0,)), ((), ())),
                         preferred_element_type=jnp.float32) + b2_ref[0]
    h2 = _gelu_exact(h2).astype(jnp.bfloat16)
    y = lax.dot_general(h2, w3_ref[0], (((1,), (0,)), ((), ())),
                        preferred_element_type=jnp.float32) + b3_ref[0]
    lane = lax.broadcasted_iota(jnp.int32, (TS, EP), 1)
    wcol = jnp.sum(jnp.where(lane == e, wsel_ref[...], 0.0), axis=1, keepdims=True)
    contrib = wcol * y

    @pl.when(e == 0)
    def _():
        out_ref[pl.ds(t * TS, TS), :] = contrib

    @pl.when(e != 0)
    def _():
        out_ref[pl.ds(t * TS, TS), :] = out_ref[pl.ds(t * TS, TS), :] + contrib


def _dense_experts(x2d, w1b, b1r, w2b, b2r, w3b, b3r, wsel):
    return pl.pallas_call(
        _dense_body,
        grid=(E, NT_DENSE),
        in_specs=[
            pl.BlockSpec((TS, H), lambda e, t: (t, 0)),
            pl.BlockSpec((1, H, F), lambda e, t: (e, 0, 0)),
            pl.BlockSpec((1, 1, F), lambda e, t: (e, 0, 0)),
            pl.BlockSpec((1, F, F2), lambda e, t: (e, 0, 0)),
            pl.BlockSpec((1, 1, F2), lambda e, t: (e, 0, 0)),
            pl.BlockSpec((1, F2, H), lambda e, t: (e, 0, 0)),
            pl.BlockSpec((1, 1, H), lambda e, t: (e, 0, 0)),
            pl.BlockSpec((TS, EP), lambda e, t: (t, 0)),
        ],
        out_specs=pl.BlockSpec((S, H), lambda e, t: (0, 0)),
        out_shape=jax.ShapeDtypeStruct((S, H), jnp.float32),
    )(x2d, w1b, b1r, w2b, b2r, w3b, b3r, wsel)


def kernel(x, Wr, br, W1, b1, W2, b2, W3, b3):
    x2d = x.reshape(S, H)
    wrp = jnp.zeros((H, EP), jnp.float32).at[:, :E].set(Wr)
    brp = jnp.full((8, EP), NEG, jnp.float32).at[:, :E].set(br[None, :])

    w2, sel2, div, wsel, _pos = _routing(x2d, wrp, brp)

    w1b = W1.astype(jnp.bfloat16)
    w2b = W2.astype(jnp.bfloat16)
    w3b = W3.astype(jnp.bfloat16)
    out2d = _dense_experts(x2d, w1b, b1.reshape(E, 1, F), w2b, b2.reshape(E, 1, F2),
                           w3b, b3.reshape(E, 1, H), wsel)

    return (out2d.reshape(1, S, H), w2.reshape(1, S, K), sel2.reshape(1, S, K),
            div.reshape(()))


# speedup vs baseline: 3.9305x; 1.4745x over previous
import functools

import jax
import jax.numpy as jnp
from jax import lax
from jax.experimental import pallas as pl
from jax.experimental.pallas import tpu as pltpu
from jax.experimental.pallas import tpu_sc as plsc

S, H, F, E, K = 2048, 768, 3072, 8, 2
F2 = F // 2
EP = 128
TS = 256
CAP = 2 * S + E * TS
NT = CAP // TS
NTP = 32
NC, NS = 2, 16
NW = NC * NS
TPW = S // NW
NEG = -1e30


def _gelu_exact(h):
    return 0.5 * h * (1.0 + lax.erf(h * 0.7071067811865476))


def _routing_body(x_ref, wr_ref, br_ref,
                  w2_ref, sel2_ref, div_ref, pos_ref, gidx_ref, vcnt_ref):
    xb = x_ref[...]
    logits = lax.dot_general(
        xb, wr_ref[...], (((1,), (0,)), ((), ())),
        preferred_element_type=jnp.float32)
    logits = logits + br_ref[0:1, :]

    lane = lax.broadcasted_iota(jnp.int32, (S, EP), 1)
    v0 = jnp.max(logits, axis=1, keepdims=True)
    e0 = jnp.min(jnp.where(logits == v0, lane, EP), axis=1, keepdims=True)
    l1 = jnp.where(lane == e0, NEG, logits)
    v1 = jnp.max(l1, axis=1, keepdims=True)
    e1 = jnp.min(jnp.where(l1 == v1, lane, EP), axis=1, keepdims=True)

    w0 = 1.0 / (1.0 + jnp.exp(v1 - v0))
    w1 = 1.0 - w0
    sel2_ref[...] = jnp.concatenate([e0, e1], axis=1)
    w2_ref[...] = jnp.concatenate([w0, w1], axis=1)

    p = jnp.exp(logits - v0)
    p = p / jnp.sum(p, axis=1, keepdims=True)
    avg = jnp.mean(p, axis=0, keepdims=True)
    ent = -jnp.sum(avg * jnp.log(avg + 1e-8))
    max_ent = jnp.log(float(E))
    div_ref[0, 0] = (max_ent - ent) / max_ent

    oh0 = (lane == e0)
    oh1 = (lane == e1)
    oh = jnp.concatenate(
        [oh0.astype(jnp.bfloat16), oh1.astype(jnp.bfloat16)], axis=0)
    r_i = lax.broadcasted_iota(jnp.int32, (TS, TS), 0)
    c_i = lax.broadcasted_iota(jnp.int32, (TS, TS), 1)
    tri = (c_i <= r_i).astype(jnp.bfloat16)
    nch = (2 * S) // TS
    cums, tots = [], []
    for c in range(nch):
        cw = lax.dot_general(tri, oh[c * TS:(c + 1) * TS, :],
                             (((1,), (0,)), ((), ())),
                             preferred_element_type=jnp.float32)
        cums.append(cw)
        tots.append(cw[TS - 1:TS, :])
    tot = jnp.concatenate(tots, axis=0)
    rc = lax.broadcasted_iota(jnp.int32, (nch, nch), 0)
    cc = lax.broadcasted_iota(jnp.int32, (nch, nch), 1)
    tri_x = (cc < rc).astype(jnp.bfloat16)
    choff = lax.dot_general(tri_x, tot.astype(jnp.bfloat16),
                            (((1,), (0,)), ((), ())),
                            preferred_element_type=jnp.float32)
    counts = jnp.sum(tot, axis=0, keepdims=True)
    padded = jnp.floor((counts + (TS - 1.0)) / TS) * TS
    rU = lax.broadcasted_iota(jnp.int32, (EP, EP), 0)
    cU = lax.broadcasted_iota(jnp.int32, (EP, EP), 1)
    U = (rU < cU).astype(jnp.bfloat16)
    offs = lax.dot_general(padded.astype(jnp.bfloat16), U,
                           (((1,), (0,)), ((), ())),
                           preferred_element_type=jnp.float32)

    csum = jnp.concatenate(
        [cums[c] + choff[c:c + 1, :] for c in range(nch)], axis=0)
    ohf = oh.astype(jnp.float32)
    rank = jnp.sum(ohf * csum, axis=1, keepdims=True) - 1.0
    ofa = jnp.sum(ohf * offs, axis=1, keepdims=True)
    pos_ref[...] = (ofa + rank).astype(jnp.int32)

    lane32 = lax.broadcasted_iota(jnp.int32, (NTP, EP), 1)
    tstart = (lax.broadcasted_iota(jnp.int32, (NTP, EP), 0) * TS).astype(jnp.float32)
    grp_end = offs + padded
    before = jnp.where((lane32 < E) & (tstart >= grp_end), 1.0, 0.0)
    gidx = jnp.minimum(jnp.sum(before, axis=1, keepdims=True),
                       float(E - 1)).astype(jnp.int32)
    ohg = (lane32 == gidx)
    counts_g = jnp.sum(jnp.where(ohg, counts, 0.0), axis=1, keepdims=True)
    offs_g = jnp.sum(jnp.where(ohg, offs, 0.0), axis=1, keepdims=True)
    tstart0 = tstart[:, 0:1]
    vcnt = jnp.clip(counts_g - (tstart0 - offs_g), 0.0, float(TS))
    gidx_ref[...] = gidx
    vcnt_ref[...] = vcnt.astype(jnp.int32)


def _routing(x2d, wrp, brp):
    return pl.pallas_call(
        _routing_body,
        out_shape=(
            jax.ShapeDtypeStruct((S, K), jnp.float32),
            jax.ShapeDtypeStruct((S, K), jnp.int32),
            jax.ShapeDtypeStruct((1, 1), jnp.float32),
            jax.ShapeDtypeStruct((2 * S, 1), jnp.int32),
            jax.ShapeDtypeStruct((NTP, 1), jnp.int32),
            jax.ShapeDtypeStruct((NTP, 1), jnp.int32),
        ),
        out_specs=(
            pl.BlockSpec(memory_space=pltpu.VMEM),
            pl.BlockSpec(memory_space=pltpu.VMEM),
            pl.BlockSpec(memory_space=pltpu.SMEM),
            pl.BlockSpec(memory_space=pltpu.VMEM),
            pl.BlockSpec(memory_space=pltpu.VMEM),
            pl.BlockSpec(memory_space=pltpu.VMEM),
        ),
    )(x2d, wrp, brp)


def _dispatch_body(x_hbm, pos0_hbm, pos1_hbm, xg_hbm, idx0_v, idx1_v, rows_v, sem):
    wid = lax.axis_index("s") * NC + lax.axis_index("c")
    base = wid * TPW
    pltpu.sync_copy(pos0_hbm.at[pl.ds(base, TPW)], idx0_v)
    pltpu.sync_copy(pos1_hbm.at[pl.ds(base, TPW)], idx1_v)
    pltpu.sync_copy(x_hbm.at[pl.ds(base, TPW)], rows_v)
    pltpu.async_copy(rows_v, xg_hbm.at[idx0_v], sem).wait()
    pltpu.async_copy(rows_v, xg_hbm.at[idx1_v], sem).wait()


def _dispatch(x2d, pos0, pos1):
    return pl.kernel(
        _dispatch_body,
        out_type=jax.ShapeDtypeStruct((CAP, H), jnp.float32),
        mesh=plsc.VectorSubcoreMesh(core_axis_name="c", subcore_axis_name="s"),
        scratch_types=[
            pltpu.VMEM((TPW,), jnp.int32),
            pltpu.VMEM((TPW,), jnp.int32),
            pltpu.VMEM((TPW, H), jnp.float32),
            pltpu.SemaphoreType.DMA,
        ],
    )(x2d, pos0, pos1)


def _gmm_body(gidx_ref, vcnt_ref, xg_ref, w1_ref, b1_ref, w2_ref, b2_ref,
              w3_ref, b3_ref, yg_ref):
    j = pl.program_id(0)
    xb = xg_ref[...].astype(jnp.bfloat16)
    h = lax.dot_general(xb, w1_ref[0], (((1,), (0,)), ((), ())),
                        preferred_element_type=jnp.float32) + b1_ref[0]
    h = _gelu_exact(h).astype(jnp.bfloat16)
    h2 = lax.dot_general(h, w2_ref[0], (((1,), (0,)), ((), ())),
                         preferred_element_type=jnp.float32) + b2_ref[0]
    h2 = _gelu_exact(h2).astype(jnp.bfloat16)
    y = lax.dot_general(h2, w3_ref[0], (((1,), (0,)), ((), ())),
                        preferred_element_type=jnp.float32) + b3_ref[0]
    row = lax.broadcasted_iota(jnp.int32, (TS, H), 0)
    yg_ref[...] = jnp.where(row < vcnt_ref[j], y, 0.0)


def _gmm(xg, gidx, vcnt, w1b, b1r, w2b, b2r, w3b, b3r):
    grid_spec = pltpu.PrefetchScalarGridSpec(
        num_scalar_prefetch=2,
        grid=(NT,),
        in_specs=[
            pl.BlockSpec((TS, H), lambda j, g, v: (j, 0)),
            pl.BlockSpec((1, H, F), lambda j, g, v: (g[j], 0, 0)),
            pl.BlockSpec((1, 1, F), lambda j, g, v: (g[j], 0, 0)),
            pl.BlockSpec((1, F, F2), lambda j, g, v: (g[j], 0, 0)),
            pl.BlockSpec((1, 1, F2), lambda j, g, v: (g[j], 0, 0)),
            pl.BlockSpec((1, F2, H), lambda j, g, v: (g[j], 0, 0)),
            pl.BlockSpec((1, 1, H), lambda j, g, v: (g[j], 0, 0)),
        ],
        out_specs=pl.BlockSpec((TS, H), lambda j, g, v: (j, 0)),
    )
    return pl.pallas_call(
        _gmm_body,
        grid_spec=grid_spec,
        out_shape=jax.ShapeDtypeStruct((CAP, H), jnp.float32),
    )(gidx, vcnt, xg, w1b, b1r, w2b, b2r, w3b, b3r)


def _combine_body(yg_hbm, pos0_hbm, pos1_hbm, g0_hbm, g1_hbm, idx_v, rows_v, sem):
    wid = lax.axis_index("s") * NC + lax.axis_index("c")
    base = wid * TPW
    pltpu.sync_copy(pos0_hbm.at[pl.ds(base, TPW)], idx_v)
    pltpu.async_copy(yg_hbm.at[idx_v], rows_v, sem).wait()
    pltpu.sync_copy(rows_v, g0_hbm.at[pl.ds(base, TPW)])
    pltpu.sync_copy(pos1_hbm.at[pl.ds(base, TPW)], idx_v)
    pltpu.async_copy(yg_hbm.at[idx_v], rows_v, sem).wait()
    pltpu.sync_copy(rows_v, g1_hbm.at[pl.ds(base, TPW)])


def _combine(yg, pos0, pos1):
    return pl.kernel(
        _combine_body,
        out_type=(jax.ShapeDtypeStruct((S, H), jnp.float32),
                  jax.ShapeDtypeStruct((S, H), jnp.float32)),
        mesh=plsc.VectorSubcoreMesh(core_axis_name="c", subcore_axis_name="s"),
        scratch_types=[
            pltpu.VMEM((TPW,), jnp.int32),
            pltpu.VMEM((TPW, H), jnp.float32),
            pltpu.SemaphoreType.DMA,
        ],
    )(yg, pos0, pos1)


def _wsum_body(g0_ref, g1_ref, w0_ref, w1_ref, out_ref):
    out_ref[...] = w0_ref[...] * g0_ref[...] + w1_ref[...] * g1_ref[...]


def _wsum(g0, g1, w0c, w1c):
    return pl.pallas_call(
        _wsum_body,
        out_shape=jax.ShapeDtypeStruct((S, H), jnp.float32),
    )(g0, g1, w0c, w1c)


def kernel(x, Wr, br, W1, b1, W2, b2, W3, b3):
    x2d = x.reshape(S, H)
    wrp = jnp.zeros((H, EP), jnp.float32).at[:, :E].set(Wr)
    brp = jnp.full((8, EP), NEG, jnp.float32).at[:, :E].set(br[None, :])

    w2, sel2, div, pos, gidx_o, vcnt_o = _routing(x2d, wrp, brp)
    pos0 = pos[:S, 0]
    pos1 = pos[S:, 0]
    gidx = gidx_o[:NT, 0]
    vcnt = vcnt_o[:NT, 0]

    xg = _dispatch(x2d, pos0, pos1)

    w1b = W1.astype(jnp.bfloat16)
    w2b = W2.astype(jnp.bfloat16)
    w3b = W3.astype(jnp.bfloat16)
    yg = _gmm(xg, gidx, vcnt, w1b, b1.reshape(E, 1, F), w2b,
              b2.reshape(E, 1, F2), w3b, b3.reshape(E, 1, H))

    g0, g1 = _combine(yg, pos0, pos1)
    out2d = _wsum(g0, g1, w2[:, 0:1], w2[:, 1:2])

    return (out2d.reshape(1, S, H), w2.reshape(1, S, K), sel2.reshape(1, S, K),
            div.reshape(()))


# speedup vs baseline: 4.3437x; 1.1051x over previous
import functools

import jax
import jax.numpy as jnp
from jax import lax
from jax.experimental import pallas as pl
from jax.experimental.pallas import tpu as pltpu
from jax.experimental.pallas import tpu_sc as plsc

S, H, F, E, K = 2048, 768, 3072, 8, 2
F2 = F // 2
EP = 128
TS = 256
CAP = 2 * S + E * TS
NT = CAP // TS
NTP = 32
NC, NS = 2, 16
NW = NC * NS
TPW = S // NW
NEG = -1e30


def _gelu_exact(h):
    return 0.5 * h * (1.0 + lax.erf(h * 0.7071067811865476))


def _routing_body(x_ref, wr_ref, br_ref,
                  w2_ref, sel2_ref, div_ref, pos_ref, gidx_ref, vcnt_ref):
    xb = x_ref[...]
    logits = lax.dot_general(
        xb, wr_ref[...], (((1,), (0,)), ((), ())),
        preferred_element_type=jnp.float32)
    logits = logits + br_ref[0:1, :]

    lane = lax.broadcasted_iota(jnp.int32, (S, EP), 1)
    v0 = jnp.max(logits, axis=1, keepdims=True)
    e0 = jnp.min(jnp.where(logits == v0, lane, EP), axis=1, keepdims=True)
    l1 = jnp.where(lane == e0, NEG, logits)
    v1 = jnp.max(l1, axis=1, keepdims=True)
    e1 = jnp.min(jnp.where(l1 == v1, lane, EP), axis=1, keepdims=True)

    w0 = 1.0 / (1.0 + jnp.exp(v1 - v0))
    w1 = 1.0 - w0
    sel2_ref[...] = jnp.concatenate([e0, e1], axis=1)
    w2_ref[...] = jnp.concatenate([w0, w1], axis=1)

    p = jnp.exp(logits - v0)
    p = p / jnp.sum(p, axis=1, keepdims=True)
    avg = jnp.mean(p, axis=0, keepdims=True)
    ent = -jnp.sum(avg * jnp.log(avg + 1e-8))
    max_ent = jnp.log(float(E))
    div_ref[0, 0] = (max_ent - ent) / max_ent

    oh0 = (lane == e0)
    oh1 = (lane == e1)
    oh = jnp.concatenate(
        [oh0.astype(jnp.bfloat16), oh1.astype(jnp.bfloat16)], axis=0)
    r_i = lax.broadcasted_iota(jnp.int32, (TS, TS), 0)
    c_i = lax.broadcasted_iota(jnp.int32, (TS, TS), 1)
    tri = (c_i <= r_i).astype(jnp.bfloat16)
    nch = (2 * S) // TS
    cums, tots = [], []
    for c in range(nch):
        cw = lax.dot_general(tri, oh[c * TS:(c + 1) * TS, :],
                             (((1,), (0,)), ((), ())),
                             preferred_element_type=jnp.float32)
        cums.append(cw)
        tots.append(cw[TS - 1:TS, :])
    tot = jnp.concatenate(tots, axis=0)
    rc = lax.broadcasted_iota(jnp.int32, (nch, nch), 0)
    cc = lax.broadcasted_iota(jnp.int32, (nch, nch), 1)
    tri_x = (cc < rc).astype(jnp.bfloat16)
    choff = lax.dot_general(tri_x, tot.astype(jnp.bfloat16),
                            (((1,), (0,)), ((), ())),
                            preferred_element_type=jnp.float32)
    counts = jnp.sum(tot, axis=0, keepdims=True)
    padded = jnp.floor((counts + (TS - 1.0)) / TS) * TS
    rU = lax.broadcasted_iota(jnp.int32, (EP, EP), 0)
    cU = lax.broadcasted_iota(jnp.int32, (EP, EP), 1)
    U = (rU < cU).astype(jnp.bfloat16)
    offs = lax.dot_general(padded.astype(jnp.bfloat16), U,
                           (((1,), (0,)), ((), ())),
                           preferred_element_type=jnp.float32)

    csum = jnp.concatenate(
        [cums[c] + choff[c:c + 1, :] for c in range(nch)], axis=0)
    ohf = oh.astype(jnp.float32)
    rank = jnp.sum(ohf * csum, axis=1, keepdims=True) - 1.0
    ofa = jnp.sum(ohf * offs, axis=1, keepdims=True)
    pos_ref[...] = (ofa + rank).astype(jnp.int32)

    lane32 = lax.broadcasted_iota(jnp.int32, (NTP, EP), 1)
    tstart = (lax.broadcasted_iota(jnp.int32, (NTP, EP), 0) * TS).astype(jnp.float32)
    grp_end = offs + padded
    before = jnp.where((lane32 < E) & (tstart >= grp_end), 1.0, 0.0)
    gidx = jnp.minimum(jnp.sum(before, axis=1, keepdims=True),
                       float(E - 1)).astype(jnp.int32)
    ohg = (lane32 == gidx)
    counts_g = jnp.sum(jnp.where(ohg, counts, 0.0), axis=1, keepdims=True)
    offs_g = jnp.sum(jnp.where(ohg, offs, 0.0), axis=1, keepdims=True)
    tstart0 = tstart[:, 0:1]
    vcnt = jnp.clip(counts_g - (tstart0 - offs_g), 0.0, float(TS))
    gidx_ref[...] = gidx
    vcnt_ref[...] = vcnt.astype(jnp.int32)


def _routing(x2d, wrp, brp):
    return pl.pallas_call(
        _routing_body,
        out_shape=(
            jax.ShapeDtypeStruct((S, K), jnp.float32),
            jax.ShapeDtypeStruct((S, K), jnp.int32),
            jax.ShapeDtypeStruct((1, 1), jnp.float32),
            jax.ShapeDtypeStruct((2 * S, 1), jnp.int32),
            jax.ShapeDtypeStruct((NTP, 1), jnp.int32),
            jax.ShapeDtypeStruct((NTP, 1), jnp.int32),
        ),
        out_specs=(
            pl.BlockSpec(memory_space=pltpu.VMEM),
            pl.BlockSpec(memory_space=pltpu.VMEM),
            pl.BlockSpec(memory_space=pltpu.SMEM),
            pl.BlockSpec(memory_space=pltpu.VMEM),
            pl.BlockSpec(memory_space=pltpu.VMEM),
            pl.BlockSpec(memory_space=pltpu.VMEM),
        ),
    )(x2d, wrp, brp)


def _dispatch_body(x_hbm, pos0_hbm, pos1_hbm, xg_hbm, idx0_v, idx1_v, rows_v, sem):
    wid = lax.axis_index("s") * NC + lax.axis_index("c")
    base = wid * TPW
    pltpu.sync_copy(pos0_hbm.at[pl.ds(base, TPW)], idx0_v)
    pltpu.sync_copy(pos1_hbm.at[pl.ds(base, TPW)], idx1_v)
    pltpu.sync_copy(x_hbm.at[pl.ds(base, TPW)], rows_v)
    pltpu.async_copy(rows_v, xg_hbm.at[idx0_v], sem).wait()
    pltpu.async_copy(rows_v, xg_hbm.at[idx1_v], sem).wait()


def _dispatch(x2d, pos0, pos1):
    return pl.kernel(
        _dispatch_body,
        out_type=jax.ShapeDtypeStruct((CAP, H), jnp.float32),
        mesh=plsc.VectorSubcoreMesh(core_axis_name="c", subcore_axis_name="s"),
        scratch_types=[
            pltpu.VMEM((TPW,), jnp.int32),
            pltpu.VMEM((TPW,), jnp.int32),
            pltpu.VMEM((TPW, H), jnp.float32),
            pltpu.SemaphoreType.DMA,
        ],
    )(x2d, pos0, pos1)


def _layer1_body(gidx_ref, xg_ref, w1_ref, b1_ref, h1_ref):
    h = lax.dot_general(xg_ref[...], w1_ref[0], (((1,), (0,)), ((), ())),
                        preferred_element_type=jnp.float32) + b1_ref[0]
    h1_ref[...] = _gelu_exact(h).astype(jnp.bfloat16)


def _layer2_body(gidx_ref, h1_ref, w2_ref, b2_ref, h2_ref):
    h = lax.dot_general(h1_ref[...].astype(jnp.float32), w2_ref[0],
                        (((1,), (0,)), ((), ())),
                        preferred_element_type=jnp.float32) + b2_ref[0]
    h2_ref[...] = _gelu_exact(h).astype(jnp.bfloat16)


def _layer3_body(gidx_ref, vcnt_ref, h2_ref, w3_ref, b3_ref, yg_ref):
    j = pl.program_id(0)
    y = lax.dot_general(h2_ref[...].astype(jnp.float32), w3_ref[0],
                        (((1,), (0,)), ((), ())),
                        preferred_element_type=jnp.float32) + b3_ref[0]
    row = lax.broadcasted_iota(jnp.int32, (TS, H), 0)
    yg_ref[...] = jnp.where(row < vcnt_ref[j], y, 0.0)


def _layer_call(body, xin, win, bin_, gidx, vcnt, din, dout, out_dtype,
                need_vcnt):
    nsp = 2 if need_vcnt else 1
    wmap = (lambda j, g, v: (g[j], 0, 0)) if need_vcnt else (
        lambda j, g: (g[j], 0, 0))
    xmap = (lambda j, g, v: (j, 0)) if need_vcnt else (lambda j, g: (j, 0))
    grid_spec = pltpu.PrefetchScalarGridSpec(
        num_scalar_prefetch=nsp,
        grid=(NT,),
        in_specs=[
            pl.BlockSpec((TS, din), xmap),
            pl.BlockSpec((1, din, dout), wmap),
            pl.BlockSpec((1, 1, dout), wmap),
        ],
        out_specs=pl.BlockSpec((TS, dout), xmap),
    )
    args = (gidx, vcnt) if need_vcnt else (gidx,)
    return pl.pallas_call(
        body,
        grid_spec=grid_spec,
        out_shape=jax.ShapeDtypeStruct((CAP, dout), out_dtype),
    )(*args, xin, win, bin_)


def _gmm(xg, gidx, vcnt, W1, b1r, W2, b2r, W3, b3r):
    h1 = _layer_call(_layer1_body, xg, W1, b1r, gidx, None, H, F,
                     jnp.bfloat16, False)
    h2 = _layer_call(_layer2_body, h1, W2, b2r, gidx, None, F, F2,
                     jnp.bfloat16, False)
    return _layer_call(_layer3_body, h2, W3, b3r, gidx, vcnt, F2, H,
                       jnp.float32, True)


def _combine_body(yg_hbm, pos0_hbm, pos1_hbm, g0_hbm, g1_hbm, idx_v, rows_v, sem):
    wid = lax.axis_index("s") * NC + lax.axis_index("c")
    base = wid * TPW
    pltpu.sync_copy(pos0_hbm.at[pl.ds(base, TPW)], idx_v)
    pltpu.async_copy(yg_hbm.at[idx_v], rows_v, sem).wait()
    pltpu.sync_copy(rows_v, g0_hbm.at[pl.ds(base, TPW)])
    pltpu.sync_copy(pos1_hbm.at[pl.ds(base, TPW)], idx_v)
    pltpu.async_copy(yg_hbm.at[idx_v], rows_v, sem).wait()
    pltpu.sync_copy(rows_v, g1_hbm.at[pl.ds(base, TPW)])


def _combine(yg, pos0, pos1):
    return pl.kernel(
        _combine_body,
        out_type=(jax.ShapeDtypeStruct((S, H), jnp.float32),
                  jax.ShapeDtypeStruct((S, H), jnp.float32)),
        mesh=plsc.VectorSubcoreMesh(core_axis_name="c", subcore_axis_name="s"),
        scratch_types=[
            pltpu.VMEM((TPW,), jnp.int32),
            pltpu.VMEM((TPW, H), jnp.float32),
            pltpu.SemaphoreType.DMA,
        ],
    )(yg, pos0, pos1)


def _wsum_body(g0_ref, g1_ref, w0_ref, w1_ref, out_ref):
    out_ref[...] = w0_ref[...] * g0_ref[...] + w1_ref[...] * g1_ref[...]


def _wsum(g0, g1, w0c, w1c):
    return pl.pallas_call(
        _wsum_body,
        out_shape=jax.ShapeDtypeStruct((S, H), jnp.float32),
    )(g0, g1, w0c, w1c)


def kernel(x, Wr, br, W1, b1, W2, b2, W3, b3):
    x2d = x.reshape(S, H)
    wrp = jnp.zeros((H, EP), jnp.float32).at[:, :E].set(Wr)
    brp = jnp.full((8, EP), NEG, jnp.float32).at[:, :E].set(br[None, :])

    w2, sel2, div, pos, gidx_o, vcnt_o = _routing(x2d, wrp, brp)
    pos0 = pos[:S, 0]
    pos1 = pos[S:, 0]
    gidx = gidx_o[:NT, 0]
    vcnt = vcnt_o[:NT, 0]

    xg = _dispatch(x2d, pos0, pos1)

    yg = _gmm(xg, gidx, vcnt, W1, b1.reshape(E, 1, F), W2,
              b2.reshape(E, 1, F2), W3, b3.reshape(E, 1, H))

    g0, g1 = _combine(yg, pos0, pos1)
    out2d = _wsum(g0, g1, w2[:, 0:1], w2[:, 1:2])

    return (out2d.reshape(1, S, H), w2.reshape(1, S, K), sel2.reshape(1, S, K),
            div.reshape(()))
